# Initial kernel scaffold; baseline (speedup 1.0000x reference)
import jax
import jax.numpy as jnp
from jax.experimental import pallas as pl


def kernel(X, time, W1, b1, W2, b2, W3, b3, Wt1, bt1, Wt2, bt2, We0, be0, We1, be1):
    raise NotImplementedError("write your pallas kernel here")


# speedup vs baseline: 8.2337x; 8.2337x over previous
import jax
import jax.numpy as jnp
from jax.experimental import pallas as pl
from jax.experimental.pallas import tpu as pltpu

_N = 128
_H = 128
_T = 128


def _dot(a, b):
    return jnp.dot(a, b, preferred_element_type=jnp.float32)


def _body(x_ref, time_ref, w1_ref, b1_ref, w2_ref, b2_ref, w3_ref, b3_ref,
          wt1_ref, bt1_ref, wt2_ref, bt2_ref, we0_ref, be0_ref, we1_ref,
          be1_ref, out_ref):
    b = pl.program_id(0)
    x = x_ref[0]

    row = jax.lax.broadcasted_iota(jnp.int32, (_N, _N), 0)
    col = jax.lax.broadcasted_iota(jnp.int32, (_N, _N), 1)
    eye = (row == col).astype(jnp.float32)
    a_hat = (x != 0.0).astype(jnp.float32) + eye
    deg = jnp.sum(a_hat, axis=1, keepdims=True)
    dinv = jax.lax.rsqrt(deg)
    adjn = dinv * a_hat * dinv.reshape(1, _N)

    h = jnp.maximum(_dot(adjn, w1_ref[...]) + b1_ref[...], 0.0)
    h = jnp.maximum(_dot(adjn, _dot(h, w2_ref[...])) + b2_ref[...], 0.0)
    h = _dot(adjn, _dot(h, w3_ref[...])) + b3_ref[...]

    we1e = we1_ref[0:_H, :]
    wa = _dot(we0_ref[0:_H, :], we1e)
    wc = _dot(we0_ref[_H:, :], we1e)
    a = _dot(h, wa)
    c = _dot(h, wc)

    t = time_ref[pl.ds(b, 1), :]
    te = jax.nn.gelu(_dot(t, wt1_ref[...]) + bt1_ref[...])
    te = _dot(te, wt2_ref[...]) + bt2_ref[...]
    d = (_dot(te, we1_ref[_H:, :])[0, 0]
         + _dot(be0_ref[...], we1e)[0, 0]
         + be1_ref[0, 0])

    out_ref[0] = a + c.reshape(1, _N) + d


def kernel(X, time, W1, b1, W2, b2, W3, b3, Wt1, bt1, Wt2, bt2,
           We0, be0, We1, be1):
    batch = X.shape[0]
    x = X.reshape(batch, _N, _N)
    b1r = b1.reshape(1, _H)
    b2r = b2.reshape(1, _H)
    b3r = b3.reshape(1, _H)
    bt1r = bt1.reshape(1, _T)
    bt2r = bt2.reshape(1, _T)
    be0r = be0.reshape(1, _H)
    be1r = be1.reshape(1, 1)

    def full(arr):
        return pl.BlockSpec(arr.shape, lambda b: (0,) * arr.ndim)

    out = pl.pallas_call(
        _body,
        grid=(batch,),
        in_specs=[
            pl.BlockSpec((1, _N, _N), lambda b: (b, 0, 0)),
            full(time), full(W1), full(b1r), full(W2), full(b2r),
            full(W3), full(b3r), full(Wt1), full(bt1r), full(Wt2),
            full(bt2r), full(We0), full(be0r), full(We1), full(be1r),
        ],
        out_specs=pl.BlockSpec((1, _N, _N), lambda b: (b, 0, 0)),
        out_shape=jax.ShapeDtypeStruct((batch, _N, _N), jnp.float32),
        compiler_params=pltpu.CompilerParams(
            dimension_semantics=("arbitrary",),
        ),
    )(x, time, W1, b1r, W2, b2r, W3, b3r, Wt1, bt1r, Wt2, bt2r,
      We0, be0r, We1, be1r)
    return out


# speedup vs baseline: 10.4324x; 1.2670x over previous
import jax
import jax.numpy as jnp
from jax.experimental import pallas as pl
from jax.experimental.pallas import tpu as pltpu

_N = 128
_H = 128
_T = 128


def _dot(a, b):
    return jnp.dot(a, b, preferred_element_type=jnp.float32)


def _body(x_ref, time_ref, w1_ref, b1_ref, w2_ref, b2_ref, w3_ref, b3_ref,
          wt1_ref, bt1_ref, wt2_ref, bt2_ref, we0_ref, be0_ref, we1_ref,
          be1_ref, out_ref, vac_ref, dvec_ref):
    b = pl.program_id(0)

    @pl.when(b == 0)
    def _init():
        we1e = we1_ref[0:_H, :]
        wa = _dot(we0_ref[0:_H, :], we1e)
        wc = _dot(we0_ref[_H:, :], we1e)
        wac = jnp.concatenate([wa, wc], axis=1)
        vac_ref[...] = _dot(w3_ref[...], wac)
        te = jax.nn.gelu(_dot(time_ref[...], wt1_ref[...]) + bt1_ref[...])
        te = _dot(te, wt2_ref[...]) + bt2_ref[...]
        b3ac = _dot(b3_ref[...], wac)
        const = (_dot(be0_ref[...], we1e)[0, 0] + be1_ref[0, 0]
                 + b3ac[0, 0] + b3ac[0, 1])
        dvec_ref[...] = _dot(te, we1_ref[_H:, :]) + const

    x = x_ref[0]
    row = jax.lax.broadcasted_iota(jnp.int32, (_N, _N), 0)
    col = jax.lax.broadcasted_iota(jnp.int32, (_N, _N), 1)
    eye = (row == col).astype(jnp.float32)
    a_hat = (x != 0.0).astype(jnp.float32) + eye
    deg = jnp.sum(a_hat, axis=1, keepdims=True)
    dinv = jax.lax.rsqrt(deg)
    adjn = dinv * a_hat * dinv.reshape(1, _N)

    h = jnp.maximum(_dot(adjn, w1_ref[...]) + b1_ref[...], 0.0)
    h = jnp.maximum(_dot(adjn, _dot(h, w2_ref[...])) + b2_ref[...], 0.0)
    pac = _dot(adjn, _dot(h, vac_ref[...]))

    d = dvec_ref[pl.ds(b, 1), :]
    out_ref[0] = pac[:, 0:1] + pac[:, 1:2].reshape(1, _N) + d


def kernel(X, time, W1, b1, W2, b2, W3, b3, Wt1, bt1, Wt2, bt2,
           We0, be0, We1, be1):
    batch = X.shape[0]
    x = X.reshape(batch, _N, _N)
    b1r = b1.reshape(1, _H)
    b2r = b2.reshape(1, _H)
    b3r = b3.reshape(1, _H)
    bt1r = bt1.reshape(1, _T)
    bt2r = bt2.reshape(1, _T)
    be0r = be0.reshape(1, _H)
    be1r = be1.reshape(1, 1)

    def full(arr):
        return pl.BlockSpec(arr.shape, lambda b: (0,) * arr.ndim)

    out = pl.pallas_call(
        _body,
        grid=(batch,),
        in_specs=[
            pl.BlockSpec((1, _N, _N), lambda b: (b, 0, 0)),
            full(time), full(W1), full(b1r), full(W2), full(b2r),
            full(W3), full(b3r), full(Wt1), full(bt1r), full(Wt2),
            full(bt2r), full(We0), full(be0r), full(We1), full(be1r),
        ],
        out_specs=pl.BlockSpec((1, _N, _N), lambda b: (b, 0, 0)),
        out_shape=jax.ShapeDtypeStruct((batch, _N, _N), jnp.float32),
        scratch_shapes=[
            pltpu.VMEM((_H, 2), jnp.float32),
            pltpu.VMEM((batch, 1), jnp.float32),
        ],
        compiler_params=pltpu.CompilerParams(
            dimension_semantics=("arbitrary",),
        ),
    )(x, time, W1, b1r, W2, b2r, W3, b3r, Wt1, bt1r, Wt2, bt2r,
      We0, be0r, We1, be1r)
    return out
